# baseline (speedup 1.0000x reference)
import jax
import jax.numpy as jnp
from jax import lax
from jax.experimental import pallas as pl
from jax.experimental.pallas import tpu as pltpu
from jax.experimental.pallas import tpu_sc as plsc

B, S, H = 64, 2048, 768
E, HID, L = 64, 512, 4

BB = 8
SB = 512
K = 4


def _routing_body(species_hbm, meta_hbm, idx_v, pres_v, meta_v):
    first = (lax.axis_index("c") == 0) & (lax.axis_index("s") == 0)

    @pl.when(first)
    def _():
        pltpu.sync_copy(species_hbm, idx_v)
        zeros = jnp.zeros((16,), jnp.int32)
        ones = jnp.ones((16,), jnp.int32)
        for j in range(E // 16):
            pres_v[pl.ds(j * 16, 16)] = zeros
        mx = jnp.int32(0)
        for j in range(B // 16):
            v = idx_v[pl.ds(j * 16, 16)]
            plsc.store_scatter(pres_v, [v], ones)
            mx = jnp.maximum(mx, jnp.max(v))
        mxv = jnp.full((16,), mx, jnp.int32)
        for j in range(E // 16):
            meta_v[pl.ds(j * 16, 16)] = mxv
        carry = jnp.int32(0)
        for j in range(E // 16):
            p = pres_v[pl.ds(j * 16, 16)]
            incl = plsc.cumsum(p)
            pos = incl - p + carry
            evec = lax.iota(jnp.int32, 16) + jnp.int32(16 * j)
            plsc.store_scatter(meta_v, [pos], evec, mask=(p == 1))
            carry = carry + jnp.sum(p)
        meta_v[pl.ds(E, 16)] = jnp.full((16,), carry, jnp.int32)
        pltpu.sync_copy(meta_v, meta_hbm)


def _route(species_idx):
    f = pl.kernel(
        _routing_body,
        compiler_params=pltpu.CompilerParams(needs_layout_passes=False),
        out_type=jax.ShapeDtypeStruct((E + 16,), jnp.int32),
        mesh=plsc.VectorSubcoreMesh(core_axis_name="c", subcore_axis_name="s"),
        scratch_types=[
            pltpu.VMEM((B,), jnp.int32),
            pltpu.VMEM((E,), jnp.int32),
            pltpu.VMEM((E + 16,), jnp.int32),
        ],
    )
    return f(species_idx)


def _pool_body(h_ref, out_ref):
    j = pl.program_id(1)
    partial = jnp.sum(h_ref[...], axis=1)

    @pl.when(j == 0)
    def _():
        out_ref[...] = partial

    @pl.when(j > 0)
    def _():
        out_ref[...] = out_ref[...] + partial

    @pl.when(j == pl.num_programs(1) - 1)
    def _():
        pooled = out_ref[...] * (1.0 / S)
        mu = jnp.mean(pooled, axis=1, keepdims=True)
        var = jnp.mean((pooled - mu) ** 2, axis=1, keepdims=True)
        out_ref[...] = (pooled - mu) * jax.lax.rsqrt(var + 1e-5)


def _mlp_body(meta_ref, species_ref, xn_ref, g_ref, b_ref, w1a_ref, w1b_ref,
              w1c_ref, w1d_ref, b1_ref, w2_ref, b2_ref, out_ref):
    i = pl.program_id(0)

    @pl.when(i == 0)
    def _():
        out_ref[...] = jnp.zeros_like(out_ref)

    xn = xn_ref[...]
    for k, w1_ref in enumerate((w1a_ref, w1b_ref, w1c_ref, w1d_ref)):
        @pl.when(i * K + k < meta_ref[E])
        def _(k=k, w1_ref=w1_ref):
            e = meta_ref[i * K + k]
            mask = species_ref[...] == e
            x = xn * g_ref[e, :, :] + b_ref[e, :, :]
            h = jnp.dot(x, w1_ref[0], preferred_element_type=jnp.float32)
            h = h + b1_ref[e, :, :]
            h = 0.5 * h * (1.0 + jax.lax.erf(h * 0.7071067811865476))
            logits = jax.lax.dot_general(
                h, w2_ref[e], (((1,), (1,)), ((), ())),
                preferred_element_type=jnp.float32)
            logits = logits + b2_ref[e, :, :]
            out_ref[...] = out_ref[...] + jnp.where(mask, logits, 0.0)


def kernel(hidden_states, species_idx, ln_g, ln_b, W1, b1, W2, b2):
    species_i32 = species_idx.astype(jnp.int32)
    meta = _route(species_i32)

    xn = pl.pallas_call(
        _pool_body,
        grid=(B // BB, S // SB),
        in_specs=[pl.BlockSpec((BB, SB, H), lambda i, j: (i, j, 0))],
        out_specs=pl.BlockSpec((BB, H), lambda i, j: (i, 0)),
        out_shape=jax.ShapeDtypeStruct((B, H), jnp.float32),
    )(hidden_states)

    species2d = jnp.broadcast_to(species_i32.reshape(B, 1), (B, L))
    w2t = jnp.swapaxes(W2, 1, 2)

    def w1_lane(k):
        return pl.BlockSpec((1, H, HID), lambda i, meta: (meta[i * K + k],
                                                          0, 0))

    grid_spec = pltpu.PrefetchScalarGridSpec(
        num_scalar_prefetch=1,
        grid=(E // K,),
        in_specs=[
            pl.BlockSpec((B, L), lambda i, meta: (0, 0)),
            pl.BlockSpec((B, H), lambda i, meta: (0, 0)),
            pl.BlockSpec((E, 1, H), lambda i, meta: (0, 0, 0)),
            pl.BlockSpec((E, 1, H), lambda i, meta: (0, 0, 0)),
            w1_lane(0),
            w1_lane(1),
            w1_lane(2),
            w1_lane(3),
            pl.BlockSpec((E, 1, HID), lambda i, meta: (0, 0, 0)),
            pl.BlockSpec((E, L, HID), lambda i, meta: (0, 0, 0)),
            pl.BlockSpec((E, 1, L), lambda i, meta: (0, 0, 0)),
        ],
        out_specs=pl.BlockSpec((B, L), lambda i, meta: (0, 0)),
    )

    logits = pl.pallas_call(
        _mlp_body,
        grid_spec=grid_spec,
        out_shape=jax.ShapeDtypeStruct((B, L), jnp.float32),
    )(meta, species2d, xn, ln_g.reshape(E, 1, H), ln_b.reshape(E, 1, H),
      W1, W1, W1, W1,
      b1.reshape(E, 1, HID), w2t, b2.reshape(E, 1, L))
    return logits
